# baseline (speedup 1.0000x reference)
import jax
import jax.numpy as jnp
from jax import lax
from jax.experimental import pallas as pl
from jax.experimental.pallas import tpu as pltpu
from jax.experimental.pallas import tpu_sc as plsc

NC = 2
NS = 16
NW = NC * NS

BATCH = 4096
SEQ = 200
D = 64
B = BATCH * SEQ
RPW = B // NW
CHUNK = 64
K = RPW // CHUNK
NBUF = 4
G = K // NBUF
L = 16


def _body(x_hbm, table_hbm, out_hbm, idx_v, pidx_v, gbuf, sbuf, gsems, ssems):
    wid = lax.axis_index("s") * NC + lax.axis_index("c")
    row0 = wid * RPW
    pltpu.sync_copy(x_hbm.at[pl.ds(row0, RPW)], idx_v)

    def mk_pidx(m, carry):
        v = idx_v[pl.ds(m * L, L)]
        pidx_v[pl.ds(m * L, L)] = lax.shift_right_logical(v, 1)
        return carry

    lax.fori_loop(0, RPW // L, mk_pidx, 0)

    lanes = lax.iota(jnp.int32, L)

    def fire_gather(j, b):
        return pltpu.async_copy(
            table_hbm.at[pidx_v.at[pl.ds(j * CHUNK, CHUNK)]],
            gbuf.at[b], gsems[b])

    def extract(j, b):
        for kg in range(CHUNK // L):
            k_vec = lanes + (kg * L)
            i_vec = idx_v[pl.ds(j * CHUNK + kg * L, L)]
            hc = lax.shift_left(lax.bitwise_and(i_vec, 1), 6)
            for w in range(D):
                v = plsc.load_gather(gbuf.at[b], [k_vec, hc + w])
                plsc.store_scatter(sbuf.at[b], [k_vec, lanes * 0 + w], v)

    def fire_scatter(j, b):
        pltpu.async_copy(sbuf.at[b],
                         out_hbm.at[pl.ds(row0 + j * CHUNK, CHUNK), :],
                         ssems[b])

    def wait_scatter(b):
        pltpu.make_async_copy(sbuf.at[b],
                              out_hbm.at[pl.ds(0, CHUNK), :], ssems[b]).wait()

    hs = [fire_gather(b, b) for b in range(NBUF)]
    for b in range(NBUF):
        hs[b].wait()
        extract(b, b)
        fire_scatter(b, b)

    def group(t, carry):
        gh = []
        for b in range(NBUF):
            wait_scatter(b)
            gh.append(fire_gather(t * NBUF + b, b))
        for b in range(NBUF):
            gh[b].wait()
            extract(t * NBUF + b, b)
            fire_scatter(t * NBUF + b, b)
        return carry

    lax.fori_loop(1, G, group, 0)
    for b in range(NBUF):
        wait_scatter(b)


@jax.jit
def _lookup(xf, table2):
    mesh = plsc.VectorSubcoreMesh(
        core_axis_name="c", subcore_axis_name="s",
        num_cores=NC, num_subcores=NS)
    f = pl.kernel(
        _body,
        out_type=jax.ShapeDtypeStruct((B, D), jnp.float32),
        mesh=mesh,
        scratch_types=[
            pltpu.VMEM((RPW,), jnp.int32),
            pltpu.VMEM((RPW,), jnp.int32),
            pltpu.VMEM((NBUF, CHUNK, 2 * D), jnp.float32),
            pltpu.VMEM((NBUF, CHUNK, D), jnp.float32),
            tuple(pltpu.SemaphoreType.DMA for _ in range(NBUF)),
            tuple(pltpu.SemaphoreType.DMA for _ in range(NBUF)),
        ],
        compiler_params=pltpu.CompilerParams(use_tc_tiling_on_sc=True,
                                             needs_layout_passes=False),
    )
    return f(xf, table2)


def kernel(x, table):
    xf = x.astype(jnp.int32).reshape(-1)
    table2 = table.reshape(500000, 2 * D)
    out = _lookup(xf, table2)
    return out.reshape(BATCH, SEQ, D)


# speedup vs baseline: 2.3116x; 2.3116x over previous
import jax
import jax.numpy as jnp
from jax import lax
from jax.experimental import pallas as pl
from jax.experimental.pallas import tpu as pltpu
from jax.experimental.pallas import tpu_sc as plsc

NC = 2
NS = 16
NW = NC * NS

BATCH = 4096
SEQ = 200
D = 64
BPW = BATCH // NW
C0, C1 = 104, SEQ - 104
NBUF = 8
CPW = 2 * BPW
G = CPW // NBUF


def _body(x_hbm, table_hbm, out_hbm, idx_v, rows, gsems, ssems):
    wid = lax.axis_index("s") * NC + lax.axis_index("c")
    row0 = wid * BPW
    pltpu.sync_copy(x_hbm.at[pl.ds(row0 * SEQ, BPW * SEQ)], idx_v)

    def fire_gather(j, b):
        r = j // 2
        c0, n = (0, C0) if b % 2 == 0 else (C0, C1)
        return pltpu.async_copy(
            table_hbm.at[idx_v.at[pl.ds(r * SEQ + c0, n)]],
            rows.at[b].at[pl.ds(0, n), :], gsems[b])

    def fire_scatter(j, b):
        r = j // 2
        c0, n = (0, C0) if b % 2 == 0 else (C0, C1)
        pltpu.async_copy(rows.at[b].at[pl.ds(0, n), :],
                         out_hbm.at[row0 + r].at[pl.ds(c0, n), :], ssems[b])

    def wait_scatter(b):
        n = C0 if b % 2 == 0 else C1
        pltpu.make_async_copy(rows.at[b].at[pl.ds(0, n), :],
                              out_hbm.at[0].at[pl.ds(0, n), :], ssems[b]).wait()

    hs = [fire_gather(b, b) for b in range(NBUF)]
    for b in range(NBUF):
        hs[b].wait()
        fire_scatter(b, b)

    def group(t, carry):
        gh = []
        for b in range(NBUF):
            wait_scatter(b)
            gh.append(fire_gather(t * NBUF + b, b))
        for b in range(NBUF):
            gh[b].wait()
            fire_scatter(t * NBUF + b, b)
        return carry

    lax.fori_loop(1, G, group, 0)
    for b in range(NBUF):
        wait_scatter(b)


@jax.jit
def _lookup(x, table):
    mesh = plsc.VectorSubcoreMesh(
        core_axis_name="c", subcore_axis_name="s",
        num_cores=NC, num_subcores=NS)
    f = pl.kernel(
        _body,
        out_type=jax.ShapeDtypeStruct((BATCH, SEQ, D), jnp.float32),
        mesh=mesh,
        scratch_types=[
            pltpu.VMEM((BPW * SEQ,), jnp.int32),
            pltpu.VMEM((NBUF, C0, D), jnp.float32),
            tuple(pltpu.SemaphoreType.DMA for _ in range(NBUF)),
            tuple(pltpu.SemaphoreType.DMA for _ in range(NBUF)),
        ],
        compiler_params=pltpu.CompilerParams(use_tc_tiling_on_sc=False),
    )
    return f(x, table)


def kernel(x, table):
    return _lookup(x.astype(jnp.int32).reshape(-1), table)
